# Initial kernel scaffold; baseline (speedup 1.0000x reference)
import jax
import jax.numpy as jnp
from jax.experimental import pallas as pl


def kernel(pred, target):
    raise NotImplementedError("write your pallas kernel here")


# speedup vs baseline: 1.0188x; 1.0188x over previous
import functools
import math

import jax
import jax.numpy as jnp
from jax import lax
from jax.experimental import pallas as pl
from jax.experimental.pallas import tpu as pltpu
from jax.experimental.pallas import tpu_sc as plsc

_V = 100000
_N = 1024
_PAD = 0
_SMOOTH = 0.1
_CONF = 1.0 - _SMOOTH
_EPS = _SMOOTH / (_V - 2)
_C0 = (_V - 2) * _EPS * math.log(_EPS) + _CONF * math.log(_CONF)

_ROWS = 256
_NRB = _N // _ROWS
_COLS = 4096
_NB = (_V + _COLS - 1) // _COLS


_NC = 2
_NS = 16
_NW = _NC * _NS
_BPW = _N // _NW


@functools.cache
def _sc_gather_kernel():
    @functools.partial(
        pl.kernel,
        mesh=plsc.VectorSubcoreMesh(core_axis_name="c", subcore_axis_name="s"),
        out_type=jax.ShapeDtypeStruct((_N,), jnp.float32),
        scratch_types=[
            pltpu.VMEM((_BPW,), jnp.int32),
            pltpu.VMEM((_BPW,), jnp.float32),
            pltpu.SemaphoreType.DMA,
        ],
    )
    def _sc_gather(flat_hbm, idx_hbm, out_hbm, idx_v, vals_v, sem):
        wid = lax.axis_index("s") * _NC + lax.axis_index("c")
        base = wid * _BPW
        pltpu.sync_copy(idx_hbm.at[pl.ds(base, _BPW)], idx_v)
        pltpu.async_copy(flat_hbm.at[idx_v], vals_v, sem).wait()
        pltpu.sync_copy(vals_v, out_hbm.at[pl.ds(base, _BPW)])

    return _sc_gather


def _loss_body(pred_ref, tgt_ref, pt_ref, out_ref, m_ref, s_ref, sv_ref, p0_ref):
    i = pl.program_id(0)
    j = pl.program_id(1)

    @pl.when(j == 0)
    def _init():
        m_ref[:] = jnp.full((_ROWS, 1), -jnp.inf, jnp.float32)
        s_ref[:] = jnp.zeros((_ROWS, 1), jnp.float32)
        sv_ref[:] = jnp.zeros((_ROWS, 1), jnp.float32)
        p0_ref[:] = pred_ref[:, 0:1]

    x = pred_ref[:]
    cols = j * _COLS + lax.broadcasted_iota(jnp.int32, (_ROWS, _COLS), 1)
    valid = cols < _V
    xm = jnp.where(valid, x, -jnp.inf)
    bmax = jnp.max(xm, axis=1, keepdims=True)
    m_old = m_ref[:]
    m_new = jnp.maximum(m_old, bmax)
    s_ref[:] = s_ref[:] * jnp.exp(m_old - m_new) + jnp.sum(
        jnp.exp(xm - m_new), axis=1, keepdims=True
    )
    m_ref[:] = m_new
    sv_ref[:] = sv_ref[:] + jnp.sum(
        jnp.where(valid, x, 0.0), axis=1, keepdims=True
    )

    @pl.when(j == _NB - 1)
    def _fin():
        lse = m_ref[:] + jnp.log(s_ref[:])
        s_logp = sv_ref[:] - jnp.float32(_V) * lse
        logp0 = p0_ref[:] - lse
        logpt = pt_ref[:] - lse
        row = _C0 - (_EPS * (s_logp - logp0) + (_CONF - _EPS) * logpt)
        row = jnp.where(tgt_ref[:] != _PAD, row, 0.0)
        acc = jnp.where(i == 0, jnp.zeros((1, 1), jnp.float32), out_ref[:])
        out_ref[:] = acc + (jnp.sum(row) / _N).reshape(1, 1)


def kernel(pred, target):
    tgt = target.astype(jnp.int32)
    flat = pred.reshape(-1)
    idx = jnp.arange(_N, dtype=jnp.int32) * _V + tgt
    pt = _sc_gather_kernel()(flat, idx)
    out = pl.pallas_call(
        _loss_body,
        grid=(_NRB, _NB),
        in_specs=[
            pl.BlockSpec((_ROWS, _COLS), lambda i, j: (i, j)),
            pl.BlockSpec((_ROWS, 1), lambda i, j: (i, 0)),
            pl.BlockSpec((_ROWS, 1), lambda i, j: (i, 0)),
        ],
        out_specs=pl.BlockSpec((1, 1), lambda i, j: (0, 0)),
        out_shape=jax.ShapeDtypeStruct((1, 1), jnp.float32),
        scratch_shapes=[pltpu.VMEM((_ROWS, 1), jnp.float32)] * 4,
        compiler_params=pltpu.CompilerParams(
            dimension_semantics=("arbitrary", "arbitrary")
        ),
    )(pred, tgt.reshape(_N, 1), pt.reshape(_N, 1))
    return out[0, 0]


# speedup vs baseline: 1.0452x; 1.0259x over previous
import functools
import math

import jax
import jax.numpy as jnp
from jax import lax
from jax.experimental import pallas as pl
from jax.experimental.pallas import tpu as pltpu
from jax.experimental.pallas import tpu_sc as plsc

_V = 100000
_N = 1024
_PAD = 0
_SMOOTH = 0.1
_CONF = 1.0 - _SMOOTH
_EPS = _SMOOTH / (_V - 2)
_C0 = (_V - 2) * _EPS * math.log(_EPS) + _CONF * math.log(_CONF)

_ROWS = 32
_NRB = _N // _ROWS


_NC = 2
_NS = 16
_NW = _NC * _NS
_BPW = _N // _NW


@functools.cache
def _sc_gather_kernel():
    @functools.partial(
        pl.kernel,
        mesh=plsc.VectorSubcoreMesh(core_axis_name="c", subcore_axis_name="s"),
        out_type=jax.ShapeDtypeStruct((_N,), jnp.float32),
        scratch_types=[
            pltpu.VMEM((_BPW,), jnp.int32),
            pltpu.VMEM((_BPW,), jnp.float32),
            pltpu.SemaphoreType.DMA,
        ],
    )
    def _sc_gather(flat_hbm, idx_hbm, out_hbm, idx_v, vals_v, sem):
        wid = lax.axis_index("s") * _NC + lax.axis_index("c")
        base = wid * _BPW
        pltpu.sync_copy(idx_hbm.at[pl.ds(base, _BPW)], idx_v)
        pltpu.async_copy(flat_hbm.at[idx_v], vals_v, sem).wait()
        pltpu.sync_copy(vals_v, out_hbm.at[pl.ds(base, _BPW)])

    return _sc_gather


def _loss_body(pred_ref, tgt_ref, pt_ref, out_ref):
    i = pl.program_id(0)
    x = pred_ref[:]
    m = jnp.max(x, axis=1, keepdims=True)
    s = jnp.sum(jnp.exp(x - m), axis=1, keepdims=True)
    sv = jnp.sum(x, axis=1, keepdims=True)
    lse = m + jnp.log(s)
    s_logp = sv - jnp.float32(_V) * lse
    logp0 = x[:, 0:1] - lse
    logpt = pt_ref[:] - lse
    row = _C0 - (_EPS * (s_logp - logp0) + (_CONF - _EPS) * logpt)
    row = jnp.where(tgt_ref[:] != _PAD, row, 0.0)
    acc = jnp.where(i == 0, jnp.zeros((1, 1), jnp.float32), out_ref[:])
    out_ref[:] = acc + (jnp.sum(row) / _N).reshape(1, 1)


def kernel(pred, target):
    tgt = target.astype(jnp.int32)
    flat = pred.reshape(-1)
    idx = jnp.arange(_N, dtype=jnp.int32) * _V + tgt
    pt = _sc_gather_kernel()(flat, idx)
    out = pl.pallas_call(
        _loss_body,
        grid=(_NRB,),
        in_specs=[
            pl.BlockSpec((_ROWS, _V), lambda i: (i, 0)),
            pl.BlockSpec((_ROWS, 1), lambda i: (i, 0)),
            pl.BlockSpec((_ROWS, 1), lambda i: (i, 0)),
        ],
        out_specs=pl.BlockSpec((1, 1), lambda i: (0, 0)),
        out_shape=jax.ShapeDtypeStruct((1, 1), jnp.float32),
        compiler_params=pltpu.CompilerParams(
            dimension_semantics=("arbitrary",)
        ),
    )(pred, tgt.reshape(_N, 1), pt.reshape(_N, 1))
    return out[0, 0]


# speedup vs baseline: 6.2509x; 5.9806x over previous
import functools
import math

import jax
import jax.numpy as jnp
from jax import lax
from jax.experimental import pallas as pl
from jax.experimental.pallas import tpu as pltpu
from jax.experimental.pallas import tpu_sc as plsc

_V = 100000
_N = 1024
_PAD = 0
_SMOOTH = 0.1
_CONF = 1.0 - _SMOOTH
_EPS = _SMOOTH / (_V - 2)
_C0 = (_V - 2) * _EPS * math.log(_EPS) + _CONF * math.log(_CONF)

_RPB = 2048
_NB = (_V + _RPB - 1) // _RPB


_NC = 2
_NS = 16
_NW = _NC * _NS
_BPW = _N // _NW


@functools.cache
def _sc_gather_kernel():
    @functools.partial(
        pl.kernel,
        mesh=plsc.VectorSubcoreMesh(core_axis_name="c", subcore_axis_name="s"),
        out_type=jax.ShapeDtypeStruct((_N,), jnp.float32),
        scratch_types=[
            pltpu.VMEM((_BPW,), jnp.int32),
            pltpu.VMEM((_BPW, _N), jnp.float32),
            pltpu.VMEM((_BPW,), jnp.float32),
            pltpu.SemaphoreType.DMA,
        ],
    )
    def _sc_gather(predt_hbm, tgt_hbm, out_hbm, tgt_v, vals_v, out_v, sem):
        wid = lax.axis_index("s") * _NC + lax.axis_index("c")
        base = wid * _BPW
        pltpu.sync_copy(tgt_hbm.at[pl.ds(base, _BPW)], tgt_v)
        pltpu.async_copy(predt_hbm.at[tgt_v], vals_v, sem).wait()
        io = lax.iota(jnp.int32, 16)
        for k in range(_BPW // 16):
            lane0 = base + 16 * k
            acc = jnp.zeros((16,), jnp.float32)
            for i in range(16):
                row = vals_v[16 * k + i, pl.ds(lane0, 16)]
                acc = jnp.where(io == i, row, acc)
            out_v[pl.ds(16 * k, 16)] = acc
        pltpu.sync_copy(out_v, out_hbm.at[pl.ds(base, _BPW)])

    return _sc_gather


def _loss_body(predt_ref, tgt_ref, pt_ref, out_ref, m_ref, s_ref, sv_ref,
               p0_ref):
    j = pl.program_id(0)

    @pl.when(j == 0)
    def _init():
        m_ref[:] = jnp.full((1, _N), -jnp.inf, jnp.float32)
        s_ref[:] = jnp.zeros((1, _N), jnp.float32)
        sv_ref[:] = jnp.zeros((1, _N), jnp.float32)
        p0_ref[:] = predt_ref[0:1, :]

    x = predt_ref[:]
    m_old = m_ref[:]

    @pl.when(j < _NB - 1)
    def _full():
        bmax = jnp.max(x, axis=0, keepdims=True)
        m_new = jnp.maximum(m_old, bmax)
        s_ref[:] = s_ref[:] * jnp.exp(m_old - m_new) + jnp.sum(
            jnp.exp(x - m_new), axis=0, keepdims=True
        )
        m_ref[:] = m_new
        sv_ref[:] = sv_ref[:] + jnp.sum(x, axis=0, keepdims=True)

    @pl.when(j == _NB - 1)
    def _tail():
        rows = j * _RPB + lax.broadcasted_iota(jnp.int32, (_RPB, _N), 0)
        valid = rows < _V
        xm = jnp.where(valid, x, -jnp.inf)
        bmax = jnp.max(xm, axis=0, keepdims=True)
        m_new = jnp.maximum(m_old, bmax)
        s = s_ref[:] * jnp.exp(m_old - m_new) + jnp.sum(
            jnp.exp(xm - m_new), axis=0, keepdims=True
        )
        sv = sv_ref[:] + jnp.sum(jnp.where(valid, x, 0.0), axis=0,
                                 keepdims=True)
        lse = m_new + jnp.log(s)
        s_logp = sv - jnp.float32(_V) * lse
        logp0 = p0_ref[:] - lse
        logpt = pt_ref[:] - lse
        row = _C0 - (_EPS * (s_logp - logp0) + (_CONF - _EPS) * logpt)
        row = jnp.where(tgt_ref[:] != _PAD, row, 0.0)
        out_ref[:] = (jnp.sum(row) / _N).reshape(1, 1)


def kernel(pred, target):
    tgt = target.astype(jnp.int32)
    predt = pred.T
    pt = _sc_gather_kernel()(predt, tgt)
    out = pl.pallas_call(
        _loss_body,
        grid=(_NB,),
        in_specs=[
            pl.BlockSpec((_RPB, _N), lambda j: (j, 0)),
            pl.BlockSpec((1, _N), lambda j: (0, 0)),
            pl.BlockSpec((1, _N), lambda j: (0, 0)),
        ],
        out_specs=pl.BlockSpec((1, 1), lambda j: (0, 0)),
        out_shape=jax.ShapeDtypeStruct((1, 1), jnp.float32),
        scratch_shapes=[pltpu.VMEM((1, _N), jnp.float32)] * 4,
        compiler_params=pltpu.CompilerParams(
            dimension_semantics=("arbitrary",)
        ),
    )(predt, tgt.reshape(1, _N), pt.reshape(1, _N))
    return out[0, 0]


# speedup vs baseline: 6.5386x; 1.0460x over previous
import functools
import math

import jax
import jax.numpy as jnp
from jax import lax
from jax.experimental import pallas as pl
from jax.experimental.pallas import tpu as pltpu
from jax.experimental.pallas import tpu_sc as plsc

_V = 100000
_N = 1024
_PAD = 0
_SMOOTH = 0.1
_CONF = 1.0 - _SMOOTH
_EPS = _SMOOTH / (_V - 2)
_C0 = (_V - 2) * _EPS * math.log(_EPS) + _CONF * math.log(_CONF)

_RPB = 3072
_NB = (_V + _RPB - 1) // _RPB


_NC = 2
_NS = 16
_NW = _NC * _NS
_BPW = _N // _NW


@functools.cache
def _sc_gather_kernel():
    @functools.partial(
        pl.kernel,
        mesh=plsc.VectorSubcoreMesh(core_axis_name="c", subcore_axis_name="s"),
        out_type=jax.ShapeDtypeStruct((_N,), jnp.float32),
        scratch_types=[
            pltpu.VMEM((_BPW,), jnp.int32),
            pltpu.VMEM((_BPW, _N), jnp.float32),
            pltpu.VMEM((_BPW,), jnp.float32),
            pltpu.SemaphoreType.DMA,
        ],
    )
    def _sc_gather(predt_hbm, tgt_hbm, out_hbm, tgt_v, vals_v, out_v, sem):
        wid = lax.axis_index("s") * _NC + lax.axis_index("c")
        base = wid * _BPW
        pltpu.sync_copy(tgt_hbm.at[pl.ds(base, _BPW)], tgt_v)
        pltpu.async_copy(predt_hbm.at[tgt_v], vals_v, sem).wait()
        io = lax.iota(jnp.int32, 16)
        for k in range(_BPW // 16):
            lane0 = base + 16 * k
            acc = jnp.zeros((16,), jnp.float32)
            for i in range(16):
                row = vals_v[16 * k + i, pl.ds(lane0, 16)]
                acc = jnp.where(io == i, row, acc)
            out_v[pl.ds(16 * k, 16)] = acc
        pltpu.sync_copy(out_v, out_hbm.at[pl.ds(base, _BPW)])

    return _sc_gather


def _loss_body(predt_ref, tgt_ref, pt_ref, out_ref, m_ref, s_ref, sv_ref,
               p0_ref):
    j = pl.program_id(0)

    @pl.when(j == 0)
    def _init():
        m_ref[:] = jnp.full((1, _N), -jnp.inf, jnp.float32)
        s_ref[:] = jnp.zeros((1, _N), jnp.float32)
        sv_ref[:] = jnp.zeros((1, _N), jnp.float32)
        p0_ref[:] = predt_ref[0:1, :]

    x = predt_ref[:]
    m_old = m_ref[:]

    @pl.when(j < _NB - 1)
    def _full():
        bmax = jnp.max(x, axis=0, keepdims=True)
        m_new = jnp.maximum(m_old, bmax)
        s_ref[:] = s_ref[:] * jnp.exp(m_old - m_new) + jnp.sum(
            jnp.exp(x - m_new), axis=0, keepdims=True
        )
        m_ref[:] = m_new
        sv_ref[:] = sv_ref[:] + jnp.sum(x, axis=0, keepdims=True)

    @pl.when(j == _NB - 1)
    def _tail():
        rows = j * _RPB + lax.broadcasted_iota(jnp.int32, (_RPB, _N), 0)
        valid = rows < _V
        xm = jnp.where(valid, x, -jnp.inf)
        bmax = jnp.max(xm, axis=0, keepdims=True)
        m_new = jnp.maximum(m_old, bmax)
        s = s_ref[:] * jnp.exp(m_old - m_new) + jnp.sum(
            jnp.exp(xm - m_new), axis=0, keepdims=True
        )
        sv = sv_ref[:] + jnp.sum(jnp.where(valid, x, 0.0), axis=0,
                                 keepdims=True)
        lse = m_new + jnp.log(s)
        s_logp = sv - jnp.float32(_V) * lse
        logp0 = p0_ref[:] - lse
        logpt = pt_ref[:] - lse
        row = _C0 - (_EPS * (s_logp - logp0) + (_CONF - _EPS) * logpt)
        row = jnp.where(tgt_ref[:] != _PAD, row, 0.0)
        out_ref[:] = (jnp.sum(row) / _N).reshape(1, 1)


def kernel(pred, target):
    tgt = target.astype(jnp.int32)
    predt = pred.T
    pt = _sc_gather_kernel()(predt, tgt)
    out = pl.pallas_call(
        _loss_body,
        grid=(_NB,),
        in_specs=[
            pl.BlockSpec((_RPB, _N), lambda j: (j, 0)),
            pl.BlockSpec((1, _N), lambda j: (0, 0)),
            pl.BlockSpec((1, _N), lambda j: (0, 0)),
        ],
        out_specs=pl.BlockSpec((1, 1), lambda j: (0, 0)),
        out_shape=jax.ShapeDtypeStruct((1, 1), jnp.float32),
        scratch_shapes=[pltpu.VMEM((1, _N), jnp.float32)] * 4,
        compiler_params=pltpu.CompilerParams(
            dimension_semantics=("arbitrary",)
        ),
    )(predt, tgt.reshape(1, _N), pt.reshape(1, _N))
    return out[0, 0]


# speedup vs baseline: 6.6476x; 1.0167x over previous
import functools
import math

import jax
import jax.numpy as jnp
from jax import lax
from jax.experimental import pallas as pl
from jax.experimental.pallas import tpu as pltpu
from jax.experimental.pallas import tpu_sc as plsc

_V = 100000
_N = 1024
_PAD = 0
_SMOOTH = 0.1
_CONF = 1.0 - _SMOOTH
_EPS = _SMOOTH / (_V - 2)
_C0 = (_V - 2) * _EPS * math.log(_EPS) + _CONF * math.log(_CONF)

_RPB = 3584
_NB = (_V + _RPB - 1) // _RPB


_NC = 2
_NS = 16
_NW = _NC * _NS
_BPW = _N // _NW


@functools.cache
def _sc_gather_kernel():
    @functools.partial(
        pl.kernel,
        mesh=plsc.VectorSubcoreMesh(core_axis_name="c", subcore_axis_name="s"),
        out_type=jax.ShapeDtypeStruct((_N,), jnp.float32),
        scratch_types=[
            pltpu.VMEM((_BPW,), jnp.int32),
            pltpu.VMEM((_BPW, _N), jnp.float32),
            pltpu.VMEM((_BPW,), jnp.float32),
            pltpu.SemaphoreType.DMA,
        ],
    )
    def _sc_gather(predt_hbm, tgt_hbm, out_hbm, tgt_v, vals_v, out_v, sem):
        wid = lax.axis_index("s") * _NC + lax.axis_index("c")
        base = wid * _BPW
        pltpu.sync_copy(tgt_hbm.at[pl.ds(base, _BPW)], tgt_v)
        pltpu.async_copy(predt_hbm.at[tgt_v], vals_v, sem).wait()
        io = lax.iota(jnp.int32, 16)
        for k in range(_BPW // 16):
            lane0 = base + 16 * k
            acc = jnp.zeros((16,), jnp.float32)
            for i in range(16):
                row = vals_v[16 * k + i, pl.ds(lane0, 16)]
                acc = jnp.where(io == i, row, acc)
            out_v[pl.ds(16 * k, 16)] = acc
        pltpu.sync_copy(out_v, out_hbm.at[pl.ds(base, _BPW)])

    return _sc_gather


def _loss_body(predt_ref, tgt_ref, pt_ref, out_ref, m_ref, s_ref, sv_ref,
               p0_ref):
    j = pl.program_id(0)

    @pl.when(j == 0)
    def _init():
        m_ref[:] = jnp.full((1, _N), -jnp.inf, jnp.float32)
        s_ref[:] = jnp.zeros((1, _N), jnp.float32)
        sv_ref[:] = jnp.zeros((1, _N), jnp.float32)
        p0_ref[:] = predt_ref[0:1, :]

    x = predt_ref[:]
    m_old = m_ref[:]

    @pl.when(j < _NB - 1)
    def _full():
        bmax = jnp.max(x, axis=0, keepdims=True)
        m_new = jnp.maximum(m_old, bmax)
        s_ref[:] = s_ref[:] * jnp.exp(m_old - m_new) + jnp.sum(
            jnp.exp(x - m_new), axis=0, keepdims=True
        )
        m_ref[:] = m_new
        sv_ref[:] = sv_ref[:] + jnp.sum(x, axis=0, keepdims=True)

    @pl.when(j == _NB - 1)
    def _tail():
        rows = j * _RPB + lax.broadcasted_iota(jnp.int32, (_RPB, _N), 0)
        valid = rows < _V
        xm = jnp.where(valid, x, -jnp.inf)
        bmax = jnp.max(xm, axis=0, keepdims=True)
        m_new = jnp.maximum(m_old, bmax)
        s = s_ref[:] * jnp.exp(m_old - m_new) + jnp.sum(
            jnp.exp(xm - m_new), axis=0, keepdims=True
        )
        sv = sv_ref[:] + jnp.sum(jnp.where(valid, x, 0.0), axis=0,
                                 keepdims=True)
        lse = m_new + jnp.log(s)
        s_logp = sv - jnp.float32(_V) * lse
        logp0 = p0_ref[:] - lse
        logpt = pt_ref[:] - lse
        row = _C0 - (_EPS * (s_logp - logp0) + (_CONF - _EPS) * logpt)
        row = jnp.where(tgt_ref[:] != _PAD, row, 0.0)
        out_ref[:] = (jnp.sum(row) / _N).reshape(1, 1)


def kernel(pred, target):
    tgt = target.astype(jnp.int32)
    predt = pred.T
    pt = _sc_gather_kernel()(predt, tgt)
    out = pl.pallas_call(
        _loss_body,
        grid=(_NB,),
        in_specs=[
            pl.BlockSpec((_RPB, _N), lambda j: (j, 0)),
            pl.BlockSpec((1, _N), lambda j: (0, 0)),
            pl.BlockSpec((1, _N), lambda j: (0, 0)),
        ],
        out_specs=pl.BlockSpec((1, 1), lambda j: (0, 0)),
        out_shape=jax.ShapeDtypeStruct((1, 1), jnp.float32),
        scratch_shapes=[pltpu.VMEM((1, _N), jnp.float32)] * 4,
        compiler_params=pltpu.CompilerParams(
            dimension_semantics=("arbitrary",)
        ),
    )(predt, tgt.reshape(1, _N), pt.reshape(1, _N))
    return out[0, 0]
